# Initial kernel scaffold; baseline (speedup 1.0000x reference)
import jax
import jax.numpy as jnp
from jax.experimental import pallas as pl


def kernel(byte_ids, emb_m, W1, b1, w2, b2, emb_v, W_cls, b_cls):
    raise NotImplementedError("write your pallas kernel here")


# speedup vs baseline: 25.5382x; 25.5382x over previous
import functools

import jax
import jax.numpy as jnp
from jax import lax
from jax.experimental import pallas as pl
from jax.experimental.pallas import tpu as pltpu
from jax.experimental.pallas import tpu_sc as plsc

_B, _L = 64, 2048
_V = 256
_MIN_KEPT = 4

_info = plsc.get_sparse_core_info()
_NC, _NS, _LANES = _info.num_cores, _info.num_subcores, _info.num_lanes
_NW = _NC * _NS
_ROWS_PER_W = _B // _NW
_CHUNKS = _L // _LANES


def _tables_body(emb_ref, w1_ref, b1_ref, w2_ref, b2_ref, logit_ref):
    h = jnp.dot(emb_ref[...], w1_ref[...], preferred_element_type=jnp.float32)
    h = jnp.maximum(h + b1_ref[...], 0.0)
    logit = jnp.dot(h, w2_ref[...], preferred_element_type=jnp.float32)
    logit_ref[...] = logit + b2_ref[0, 0]


def _masker_tables(emb_m, W1, b1, w2, b2):
    return pl.pallas_call(
        _tables_body,
        out_shape=jax.ShapeDtypeStruct((_V, 1), jnp.float32),
    )(emb_m, W1, b1.reshape(1, -1), w2.reshape(-1, 1), b2.reshape(1, 1))


def _sc_body(ids_hbm, lt_hbm, pt_hbm,
             ml_hbm, pp_hbm, hist_hbm,
             ids_v, lt_v, pt_v, ml_v, pp_v, h16_v, hist_v):
    wid = lax.axis_index("s") * _NC + lax.axis_index("c")
    pltpu.sync_copy(lt_hbm, lt_v)
    pltpu.sync_copy(pt_hbm, pt_v)
    lane_off = lax.iota(jnp.int32, _LANES) * _V
    ones = jnp.full((_LANES,), 1.0, jnp.float32)
    zeros = jnp.zeros((_LANES,), jnp.float32)

    for r in range(_ROWS_PER_W):
        row = wid * _ROWS_PER_W + r
        pltpu.sync_copy(ids_hbm.at[row], ids_v)

        def zbody(j, _):
            h16_v[pl.ds(j * _LANES, _LANES)] = zeros
            return 0
        lax.fori_loop(0, (_LANES * _V) // _LANES, zbody, 0)

        def gbody(i, _):
            base = i * _LANES
            idx = ids_v[pl.ds(base, _LANES)]
            ml_v[pl.ds(base, _LANES)] = plsc.load_gather(lt_v, [idx])
            pp_v[pl.ds(base, _LANES)] = plsc.load_gather(pt_v, [idx])
            plsc.addupdate_scatter(h16_v, [idx + lane_off], ones)
            return 0
        lax.fori_loop(0, _CHUNKS, gbody, 0)

        def rbody(j, _):
            acc = zeros
            for l in range(_LANES):
                acc = acc + h16_v[pl.ds(l * _V + j * _LANES, _LANES)]
            hist_v[pl.ds(j * _LANES, _LANES)] = acc
            return 0
        lax.fori_loop(0, _V // _LANES, rbody, 0)

        pltpu.sync_copy(ml_v, ml_hbm.at[row])
        pltpu.sync_copy(pp_v, pp_hbm.at[row])
        pltpu.sync_copy(hist_v, hist_hbm.at[row])


_sc_stage = functools.partial(
    pl.kernel,
    out_type=[
        jax.ShapeDtypeStruct((_B, _L), jnp.float32),
        jax.ShapeDtypeStruct((_B, _L), jnp.float32),
        jax.ShapeDtypeStruct((_B, _V), jnp.float32),
    ],
    mesh=plsc.VectorSubcoreMesh(core_axis_name="c", subcore_axis_name="s"),
    compiler_params=pltpu.CompilerParams(needs_layout_passes=False),
    scratch_types=[
        pltpu.VMEM((_L,), jnp.int32),
        pltpu.VMEM((_V,), jnp.float32),
        pltpu.VMEM((_V,), jnp.float32),
        pltpu.VMEM((_L,), jnp.float32),
        pltpu.VMEM((_L,), jnp.float32),
        pltpu.VMEM((_LANES * _V,), jnp.float32),
        pltpu.VMEM((_V,), jnp.float32),
    ],
)(_sc_body)


def _final_body(hist_ref, keep_ref, pc_ref, pr_ref, embv_ref, wcls_ref, bcls_ref,
                cls_ref, len_ref):
    hist = hist_ref[...]
    kept = hist * keep_ref[...]
    count = jnp.sum(kept, axis=1, keepdims=True)
    pc = pc_ref[...]
    pr = pr_ref[...]
    iu = lax.broadcasted_iota(jnp.int32, (_V, _V), 0)
    iv = lax.broadcasted_iota(jnp.int32, (_V, _V), 1)
    before = jnp.where((pc > pr) | ((pc == pr) & (iu < iv)), 1.0, 0.0)
    cum = jnp.dot(hist, before, preferred_element_type=jnp.float32,
                  precision=lax.Precision.HIGHEST)
    topk_take = jnp.minimum(jnp.maximum(float(_MIN_KEPT) - cum, 0.0), hist)
    use_fb = count < float(_MIN_KEPT)
    w = jnp.where(use_fb, topk_take, kept)
    pooled_sum = jnp.dot(w, embv_ref[...], preferred_element_type=jnp.float32,
                         precision=lax.Precision.HIGHEST)
    lengths = jnp.where(use_fb, float(_MIN_KEPT), count)
    pooled = pooled_sum / jnp.maximum(lengths, 1.0)
    cls = jnp.dot(pooled, wcls_ref[...], preferred_element_type=jnp.float32,
                  precision=lax.Precision.HIGHEST)
    cls_ref[...] = cls + bcls_ref[0, 0]
    len_ref[...] = lengths.astype(jnp.int32)


def _finalize(hist, keep_row, prob_col, prob_row, emb_v_tail, W_cls_tail, b_cls):
    return pl.pallas_call(
        _final_body,
        out_shape=[
            jax.ShapeDtypeStruct((_B, 1), jnp.float32),
            jax.ShapeDtypeStruct((_B, 1), jnp.int32),
        ],
    )(hist, keep_row, prob_col, prob_row, emb_v_tail, W_cls_tail, b_cls)


def kernel(byte_ids, emb_m, W1, b1, w2, b2, emb_v, W_cls, b_cls):
    ids = jnp.asarray(byte_ids).astype(jnp.int32)
    logit_col = _masker_tables(emb_m, W1, b1, w2, b2)
    logit_tab = logit_col.reshape(_V)
    prob_tab = jax.nn.sigmoid(logit_tab)
    keep_row = (prob_tab > 0.5).astype(jnp.float32).reshape(1, _V)

    mask_logits, prune_probs, hist = _sc_stage(ids, logit_tab, prob_tab)

    cls, lengths = _finalize(
        hist, keep_row, prob_tab.reshape(_V, 1), prob_tab.reshape(1, _V),
        emb_v, W_cls[2:, :], b_cls.reshape(1, 1))
    return mask_logits, prune_probs, cls, lengths.reshape(_B)


# speedup vs baseline: 25.5534x; 1.0006x over previous
import functools

import jax
import jax.numpy as jnp
from jax import lax
from jax.experimental import pallas as pl
from jax.experimental.pallas import tpu as pltpu
from jax.experimental.pallas import tpu_sc as plsc

_B, _L = 64, 2048
_V = 256
_MIN_KEPT = 4

_info = plsc.get_sparse_core_info()
_NC, _NS, _LANES = _info.num_cores, _info.num_subcores, _info.num_lanes
_NW = _NC * _NS
_ROWS_PER_W = _B // _NW
_CHUNKS = _L // _LANES


def _tables_body(emb_ref, w1_ref, b1_ref, w2_ref, b2_ref, logit_ref):
    h = jnp.dot(emb_ref[...], w1_ref[...], preferred_element_type=jnp.float32)
    h = jnp.maximum(h + b1_ref[...], 0.0)
    logit = jnp.dot(h, w2_ref[...], preferred_element_type=jnp.float32)
    logit_ref[...] = logit + b2_ref[0, 0]


def _masker_tables(emb_m, W1, b1, w2, b2):
    return pl.pallas_call(
        _tables_body,
        out_shape=jax.ShapeDtypeStruct((_V, 1), jnp.float32),
    )(emb_m, W1, b1.reshape(1, -1), w2.reshape(-1, 1), b2.reshape(1, 1))


def _sc_body(ids_hbm, lt_hbm, pt_hbm,
             ml_hbm, pp_hbm, hist_hbm,
             ids_v, lt_v, pt_v, ml_v, pp_v, h16_v, hist_v):
    wid = lax.axis_index("s") * _NC + lax.axis_index("c")
    pltpu.sync_copy(lt_hbm, lt_v)
    pltpu.sync_copy(pt_hbm, pt_v)
    lane_off = lax.iota(jnp.int32, _LANES) * _V
    ones = jnp.full((_LANES,), 1.0, jnp.float32)
    zeros = jnp.zeros((_LANES,), jnp.float32)

    for r in range(_ROWS_PER_W):
        row = wid * _ROWS_PER_W + r
        pltpu.sync_copy(ids_hbm.at[row], ids_v)

        def zbody(j, _):
            h16_v[pl.ds(j * _LANES, _LANES)] = zeros
            return 0
        lax.fori_loop(0, (_LANES * _V) // _LANES, zbody, 0)

        def gbody(i, _):
            base = i * _LANES
            idx = ids_v[pl.ds(base, _LANES)]
            ml_v[pl.ds(base, _LANES)] = plsc.load_gather(lt_v, [idx])
            pp_v[pl.ds(base, _LANES)] = plsc.load_gather(pt_v, [idx])
            plsc.addupdate_scatter(h16_v, [idx + lane_off], ones)
            return 0
        lax.fori_loop(0, _CHUNKS, gbody, 0)

        def rbody(j, _):
            acc = zeros
            for l in range(_LANES):
                acc = acc + h16_v[pl.ds(l * _V + j * _LANES, _LANES)]
            hist_v[pl.ds(j * _LANES, _LANES)] = acc
            return 0
        lax.fori_loop(0, _V // _LANES, rbody, 0)

        pltpu.sync_copy(ml_v, ml_hbm.at[row])
        pltpu.sync_copy(pp_v, pp_hbm.at[row])
        pltpu.sync_copy(hist_v, hist_hbm.at[row])


_sc_stage = functools.partial(
    pl.kernel,
    out_type=[
        jax.ShapeDtypeStruct((_B, _L), jnp.float32),
        jax.ShapeDtypeStruct((_B, _L), jnp.float32),
        jax.ShapeDtypeStruct((_B, _V), jnp.float32),
    ],
    mesh=plsc.VectorSubcoreMesh(core_axis_name="c", subcore_axis_name="s"),
    compiler_params=pltpu.CompilerParams(needs_layout_passes=False),
    scratch_types=[
        pltpu.VMEM((_L,), jnp.int32),
        pltpu.VMEM((_V,), jnp.float32),
        pltpu.VMEM((_V,), jnp.float32),
        pltpu.VMEM((_L,), jnp.float32),
        pltpu.VMEM((_L,), jnp.float32),
        pltpu.VMEM((_LANES * _V,), jnp.float32),
        pltpu.VMEM((_V,), jnp.float32),
    ],
)(_sc_body)


def _final_body(hist_ref, keep_ref, pc_ref, pr_ref, embv_ref, wcls_ref, bcls_ref,
                cls_ref, len_ref):
    hist = hist_ref[...]
    kept = hist * keep_ref[...]
    count = jnp.sum(kept, axis=1, keepdims=True)
    pc = pc_ref[...]
    pr = pr_ref[...]
    iu = lax.broadcasted_iota(jnp.int32, (_V, _V), 0)
    iv = lax.broadcasted_iota(jnp.int32, (_V, _V), 1)
    before = jnp.where((pc > pr) | ((pc == pr) & (iu < iv)), 1.0, 0.0)
    cum = jnp.dot(hist, before, preferred_element_type=jnp.float32,
                  precision=lax.Precision.HIGHEST)
    topk_take = jnp.minimum(jnp.maximum(float(_MIN_KEPT) - cum, 0.0), hist)
    use_fb = count < float(_MIN_KEPT)
    w = jnp.where(use_fb, topk_take, kept)
    pooled_sum = jnp.dot(w, embv_ref[...], preferred_element_type=jnp.float32,
                         precision=lax.Precision.HIGHEST)
    lengths = jnp.where(use_fb, float(_MIN_KEPT), count)
    pooled = pooled_sum / jnp.maximum(lengths, 1.0)
    cls = jnp.dot(pooled, wcls_ref[...], preferred_element_type=jnp.float32)
    cls_ref[...] = cls + bcls_ref[0, 0]
    len_ref[...] = lengths.astype(jnp.int32)


def _finalize(hist, keep_row, prob_col, prob_row, emb_v_tail, W_cls_tail, b_cls):
    return pl.pallas_call(
        _final_body,
        out_shape=[
            jax.ShapeDtypeStruct((_B, 1), jnp.float32),
            jax.ShapeDtypeStruct((_B, 1), jnp.int32),
        ],
    )(hist, keep_row, prob_col, prob_row, emb_v_tail, W_cls_tail, b_cls)


def kernel(byte_ids, emb_m, W1, b1, w2, b2, emb_v, W_cls, b_cls):
    ids = jnp.asarray(byte_ids).astype(jnp.int32)
    logit_col = _masker_tables(emb_m, W1, b1, w2, b2)
    logit_tab = logit_col.reshape(_V)
    prob_tab = jax.nn.sigmoid(logit_tab)
    keep_row = (prob_tab > 0.5).astype(jnp.float32).reshape(1, _V)

    mask_logits, prune_probs, hist = _sc_stage(ids, logit_tab, prob_tab)

    cls, lengths = _finalize(
        hist, keep_row, prob_tab.reshape(_V, 1), prob_tab.reshape(1, _V),
        emb_v, W_cls[2:, :], b_cls.reshape(1, 1))
    return mask_logits, prune_probs, cls, lengths.reshape(_B)


# speedup vs baseline: 29.2412x; 1.1443x over previous
import functools

import jax
import jax.numpy as jnp
from jax import lax
from jax.experimental import pallas as pl
from jax.experimental.pallas import tpu as pltpu
from jax.experimental.pallas import tpu_sc as plsc

_B, _L = 64, 2048
_V = 256
_MIN_KEPT = 4

_info = plsc.get_sparse_core_info()
_NC, _NS, _LANES = _info.num_cores, _info.num_subcores, _info.num_lanes
_NW = _NC * _NS
_ROWS_PER_W = _B // _NW
_CHUNKS = _L // _LANES
_HB = _LANES * _V


def _tables_body(emb_ref, w1_ref, b1_ref, w2_ref, b2_ref, logit_ref):
    h = jnp.dot(emb_ref[...], w1_ref[...], preferred_element_type=jnp.float32)
    h = jnp.maximum(h + b1_ref[...], 0.0)
    logit = jnp.dot(h, w2_ref[...], preferred_element_type=jnp.float32)
    logit_ref[...] = logit + b2_ref[0, 0]


def _masker_tables(emb_m, W1, b1, w2, b2):
    return pl.pallas_call(
        _tables_body,
        out_shape=jax.ShapeDtypeStruct((_V, 1), jnp.float32),
    )(emb_m, W1, b1.reshape(1, -1), w2.reshape(-1, 1), b2.reshape(1, 1))


def _sc_body(ids_hbm, lt_hbm, pt_hbm, zeros_hbm,
             ml_hbm, pp_hbm, h16_hbm,
             ids_v, tab_v, ml_v, pp_v, h16_v,
             sem_misc, sem_r0, sem_r1, sem_out):
    wid = lax.axis_index("s") * _NC + lax.axis_index("c")
    row0 = wid * _ROWS_PER_W

    cp_tab = pltpu.async_copy(lt_hbm, tab_v.at[pl.ds(0, _V)], sem_misc)
    cp_tab2 = pltpu.async_copy(pt_hbm, tab_v.at[pl.ds(_V, _V)], sem_misc)
    cp_zero = pltpu.async_copy(zeros_hbm, h16_v, sem_misc)
    sems_r = [sem_r0, sem_r1]
    cp_ids = [
        pltpu.async_copy(ids_hbm.at[row0 + r], ids_v.at[pl.ds(r * _L, _L)],
                         sems_r[r])
        for r in range(_ROWS_PER_W)
    ]
    cp_tab.wait()
    cp_tab2.wait()
    cp_zero.wait()

    lane_off = lax.iota(jnp.int32, _LANES) * _V
    ones = jnp.full((_LANES,), 1.0, jnp.float32)
    out_cps = []
    for r in range(_ROWS_PER_W):
        cp_ids[r].wait()
        ib, hb = r * _L, r * _HB

        @functools.partial(plsc.parallel_loop, 0, _CHUNKS, unroll=8)
        def gather_body(i, _ib=ib):
            base = i * _LANES
            idx = ids_v[pl.ds(_ib + base, _LANES)]
            ml_v[pl.ds(_ib + base, _LANES)] = plsc.load_gather(
                tab_v, [idx])
            pp_v[pl.ds(_ib + base, _LANES)] = plsc.load_gather(
                tab_v, [idx + _V])

        def hist_body(j, _):
            for u in range(8):
                base = (j * 8 + u) * _LANES
                idx = ids_v[pl.ds(ib + base, _LANES)]
                plsc.addupdate_scatter(h16_v, [idx + (lane_off + hb)], ones)
            return 0
        lax.fori_loop(0, _CHUNKS // 8, hist_body, 0)

        out_cps += [
            pltpu.async_copy(ml_v.at[pl.ds(ib, _L)], ml_hbm.at[row0 + r],
                             sem_out),
            pltpu.async_copy(pp_v.at[pl.ds(ib, _L)], pp_hbm.at[row0 + r],
                             sem_out),
            pltpu.async_copy(h16_v.at[pl.ds(hb, _HB)], h16_hbm.at[row0 + r],
                             sem_out),
        ]
    for cp in out_cps:
        cp.wait()


_sc_stage = functools.partial(
    pl.kernel,
    out_type=[
        jax.ShapeDtypeStruct((_B, _L), jnp.float32),
        jax.ShapeDtypeStruct((_B, _L), jnp.float32),
        jax.ShapeDtypeStruct((_B, _HB), jnp.float32),
    ],
    mesh=plsc.VectorSubcoreMesh(core_axis_name="c", subcore_axis_name="s"),
    compiler_params=pltpu.CompilerParams(needs_layout_passes=False),
    scratch_types=[
        pltpu.VMEM((_ROWS_PER_W * _L,), jnp.int32),
        pltpu.VMEM((2 * _V,), jnp.float32),
        pltpu.VMEM((_ROWS_PER_W * _L,), jnp.float32),
        pltpu.VMEM((_ROWS_PER_W * _L,), jnp.float32),
        pltpu.VMEM((_ROWS_PER_W * _HB,), jnp.float32),
        pltpu.SemaphoreType.DMA,
        pltpu.SemaphoreType.DMA,
        pltpu.SemaphoreType.DMA,
        pltpu.SemaphoreType.DMA,
    ],
)(_sc_body)


def _final_body(h16_ref, keep_ref, pc_ref, pr_ref, embv_ref, wcls_ref, bcls_ref,
                cls_ref, len_ref):
    hist = h16_ref[:, 0:_V]
    for l in range(1, _LANES):
        hist = hist + h16_ref[:, l * _V:(l + 1) * _V]
    kept = hist * keep_ref[...]
    count = jnp.sum(kept, axis=1, keepdims=True)
    pc = pc_ref[...]
    pr = pr_ref[...]
    iu = lax.broadcasted_iota(jnp.int32, (_V, _V), 0)
    iv = lax.broadcasted_iota(jnp.int32, (_V, _V), 1)
    before = jnp.where((pc > pr) | ((pc == pr) & (iu < iv)), 1.0, 0.0)
    cum = jnp.dot(hist, before, preferred_element_type=jnp.float32,
                  precision=lax.Precision.HIGHEST)
    topk_take = jnp.minimum(jnp.maximum(float(_MIN_KEPT) - cum, 0.0), hist)
    use_fb = count < float(_MIN_KEPT)
    w = jnp.where(use_fb, topk_take, kept)
    pooled_sum = jnp.dot(w, embv_ref[...], preferred_element_type=jnp.float32,
                         precision=lax.Precision.HIGHEST)
    lengths = jnp.where(use_fb, float(_MIN_KEPT), count)
    pooled = pooled_sum / jnp.maximum(lengths, 1.0)
    cls = jnp.dot(pooled, wcls_ref[...], preferred_element_type=jnp.float32)
    cls_ref[...] = cls + bcls_ref[0, 0]
    len_ref[...] = lengths.astype(jnp.int32)


def _finalize(h16, keep_row, prob_col, prob_row, emb_v_tail, W_cls_tail, b_cls):
    return pl.pallas_call(
        _final_body,
        out_shape=[
            jax.ShapeDtypeStruct((_B, 1), jnp.float32),
            jax.ShapeDtypeStruct((_B, 1), jnp.int32),
        ],
    )(h16, keep_row, prob_col, prob_row, emb_v_tail, W_cls_tail, b_cls)


def kernel(byte_ids, emb_m, W1, b1, w2, b2, emb_v, W_cls, b_cls):
    ids = jnp.asarray(byte_ids).astype(jnp.int32)
    logit_col = _masker_tables(emb_m, W1, b1, w2, b2)
    logit_tab = logit_col.reshape(_V)
    prob_tab = jax.nn.sigmoid(logit_tab)
    keep_row = (prob_tab > 0.5).astype(jnp.float32).reshape(1, _V)
    zeros_hb = jnp.zeros((_ROWS_PER_W * _HB,), jnp.float32)

    mask_logits, prune_probs, h16 = _sc_stage(ids, logit_tab, prob_tab, zeros_hb)

    cls, lengths = _finalize(
        h16, keep_row, prob_tab.reshape(_V, 1), prob_tab.reshape(1, _V),
        emb_v, W_cls[2:, :], b_cls.reshape(1, 1))
    return mask_logits, prune_probs, cls, lengths.reshape(_B)
